# initial kernel scaffold (unmeasured)
import jax
import jax.numpy as jnp
from jax import lax
from jax.experimental import pallas as pl
from jax.experimental.pallas import tpu as pltpu

N_DEV = 4
M_PER = 1024
N_TOTAL = 8192
N_BLOCKS = 4
N_BLK = N_TOTAL // N_BLOCKS


def kernel(x, w_mat):
    x16 = x.astype(jnp.bfloat16)
    w16 = w_mat.astype(jnp.bfloat16)

    def body(x_ref, w_ref, out_ref, comm_ref, acc_ref, send_sems, recv_sems,
             out_sem):
        my_pos = lax.axis_index("i")
        left = lax.rem(my_pos + (N_DEV - 1), N_DEV)
        right = lax.rem(my_pos + 1, N_DEV)

        barrier_sem = pltpu.get_barrier_semaphore()
        for nbr in (left, right):
            pl.semaphore_signal(
                barrier_sem, inc=1,
                device_id=(nbr,), device_id_type=pl.DeviceIdType.MESH,
            )
        pl.semaphore_wait(barrier_sem, 2)

        def partial(off, col0):
            row0 = lax.rem(my_pos + off, N_DEV) * M_PER
            return jnp.dot(
                x_ref[pl.ds(row0, M_PER), :],
                w_ref[:, pl.ds(col0, N_BLK)],
                preferred_element_type=jnp.float32,
            )

        for j in range(N_BLOCKS):
            col0 = j * N_BLK
            comm_ref[0] = partial(N_DEV - 1, col0).astype(jnp.bfloat16)

            for s in range(N_DEV - 1):
                send_slot = s % 2
                recv_slot = (s + 1) % 2
                rdma = pltpu.make_async_remote_copy(
                    src_ref=comm_ref.at[send_slot],
                    dst_ref=comm_ref.at[recv_slot],
                    send_sem=send_sems.at[send_slot],
                    recv_sem=recv_sems.at[recv_slot],
                    device_id=(right,),
                    device_id_type=pl.DeviceIdType.MESH,
                )
                rdma.start()
                rdma.wait()

                p = partial(2 - s, col0)
                if s < N_DEV - 2:
                    comm_ref[recv_slot] = (
                        comm_ref[recv_slot][...].astype(jnp.float32) + p
                    ).astype(jnp.bfloat16)
                else:
                    acc_ref[...] = (
                        comm_ref[recv_slot][...].astype(jnp.float32) + p
                    )
                    cp = pltpu.make_async_copy(
                        acc_ref, out_ref.at[:, pl.ds(col0, N_BLK)], out_sem
                    )
                    cp.start()
                    cp.wait()

    out = pl.pallas_call(
        body,
        out_shape=jax.ShapeDtypeStruct((M_PER, N_TOTAL), jnp.float32),
        in_specs=[
            pl.BlockSpec(memory_space=pltpu.VMEM),
            pl.BlockSpec(memory_space=pltpu.VMEM),
        ],
        out_specs=pl.BlockSpec(memory_space=pltpu.ANY),
        scratch_shapes=[
            pltpu.VMEM((2, M_PER, N_BLK), jnp.bfloat16),
            pltpu.VMEM((M_PER, N_BLK), jnp.float32),
            pltpu.SemaphoreType.DMA((2,)),
            pltpu.SemaphoreType.DMA((2,)),
            pltpu.SemaphoreType.DMA,
        ],
        compiler_params=pltpu.CompilerParams(collective_id=0),
    )(x16, w16)
    return out


# baseline (device time: 706326 ns/iter reference)
import jax
import jax.numpy as jnp
from jax import lax
from jax.experimental import pallas as pl
from jax.experimental.pallas import tpu as pltpu

N_DEV = 4
M_PER = 1024
N_TOTAL = 8192
N_BLOCKS = 4
N_BLK = N_TOTAL // N_BLOCKS


def kernel(x, w_mat):
    x16 = x.astype(jnp.bfloat16)
    w16 = w_mat.astype(jnp.bfloat16)

    def body(x_ref, w_ref, out_ref, comm_ref, acc_ref, send_sems, recv_sems,
             out_sem):
        my_pos = lax.axis_index("i")
        left = lax.rem(my_pos + (N_DEV - 1), N_DEV)
        right = lax.rem(my_pos + 1, N_DEV)

        barrier_sem = pltpu.get_barrier_semaphore()
        for nbr in (left, right):
            pl.semaphore_signal(
                barrier_sem, inc=1,
                device_id=(nbr,), device_id_type=pl.DeviceIdType.MESH,
            )
        pl.semaphore_wait(barrier_sem, 2)

        def partial(off, col0):
            row0 = lax.rem(my_pos + off, N_DEV) * M_PER
            return jnp.dot(
                x_ref[pl.ds(row0, M_PER), :],
                w_ref[:, pl.ds(col0, N_BLK)],
                preferred_element_type=jnp.float32,
            )

        for j in range(N_BLOCKS):
            col0 = j * N_BLK
            comm_ref[0] = partial(N_DEV - 1, col0).astype(jnp.bfloat16)

            for s in range(N_DEV - 1):
                send_slot = s % 2
                recv_slot = (s + 1) % 2
                rdma = pltpu.make_async_remote_copy(
                    src_ref=comm_ref.at[send_slot],
                    dst_ref=comm_ref.at[recv_slot],
                    send_sem=send_sems.at[send_slot],
                    recv_sem=recv_sems.at[recv_slot],
                    device_id=(right,),
                    device_id_type=pl.DeviceIdType.MESH,
                )
                rdma.start()
                rdma.wait()

                p = partial(2 - s, col0)
                if s < N_DEV - 2:
                    comm_ref[recv_slot] = (
                        comm_ref[recv_slot][...].astype(jnp.float32) + p
                    ).astype(jnp.bfloat16)
                else:
                    acc_ref[...] = (
                        comm_ref[recv_slot][...].astype(jnp.float32) + p
                    )
                    cp = pltpu.make_async_copy(
                        acc_ref, out_ref.at[:, pl.ds(col0, N_BLK)], out_sem
                    )
                    cp.start()
                    cp.wait()

    out = pl.pallas_call(
        body,
        out_shape=jax.ShapeDtypeStruct((M_PER, N_TOTAL), jnp.float32),
        in_specs=[
            pl.BlockSpec(memory_space=pltpu.VMEM),
            pl.BlockSpec(memory_space=pltpu.VMEM),
        ],
        out_specs=pl.BlockSpec(memory_space=pl.ANY),
        scratch_shapes=[
            pltpu.VMEM((2, M_PER, N_BLK), jnp.bfloat16),
            pltpu.VMEM((M_PER, N_BLK), jnp.float32),
            pltpu.SemaphoreType.DMA((2,)),
            pltpu.SemaphoreType.DMA((2,)),
            pltpu.SemaphoreType.DMA,
        ],
        compiler_params=pltpu.CompilerParams(
            collective_id=0,
            vmem_limit_bytes=40 * 1024 * 1024,
        ),
    )(x16, w16)
    return out


# device time: 392060 ns/iter; 1.8016x vs baseline; 1.8016x over previous
import jax
import jax.numpy as jnp
from jax import lax
from jax.experimental import pallas as pl
from jax.experimental.pallas import tpu as pltpu

N_DEV = 4
M_PER = 1024
N_TOTAL = 8192
N_BLK = 1024
N_PHASES = 4


def kernel(x, w_mat):
    x16 = x.astype(jnp.bfloat16)
    w16 = w_mat.astype(jnp.bfloat16)

    def body(x_ref, w_ref, out_ref,
             comm_cw, comm_ccw, acc_cw, acc_ccw,
             send_cw, recv_cw, send_ccw, recv_ccw, out_sems):
        my_pos = lax.axis_index("i")
        left = lax.rem(my_pos + (N_DEV - 1), N_DEV)
        right = lax.rem(my_pos + 1, N_DEV)

        barrier_sem = pltpu.get_barrier_semaphore()
        for nbr in (left, right):
            pl.semaphore_signal(
                barrier_sem, inc=1,
                device_id=(nbr,), device_id_type=pl.DeviceIdType.MESH,
            )
        pl.semaphore_wait(barrier_sem, 2)

        def partial(off, col0):
            row0 = lax.rem(my_pos + off, N_DEV) * M_PER
            return jnp.dot(
                x_ref[pl.ds(row0, M_PER), :],
                w_ref[:, pl.ds(col0, N_BLK)],
                preferred_element_type=jnp.float32,
            )

        def hop(comm, send_sems, recv_sems, s, dst):
            return pltpu.make_async_remote_copy(
                src_ref=comm.at[s % 2],
                dst_ref=comm.at[(s + 1) % 2],
                send_sem=send_sems.at[s % 2],
                recv_sem=recv_sems.at[(s + 1) % 2],
                device_id=(dst,),
                device_id_type=pl.DeviceIdType.MESH,
            )

        comm_cw[0] = partial(3, 0).astype(jnp.bfloat16)
        comm_ccw[0] = partial(1, N_BLK).astype(jnp.bfloat16)

        for p in range(N_PHASES):
            c_cw = (2 * p) * N_BLK
            c_ccw = (2 * p + 1) * N_BLK

            for s in range(N_DEV - 1):
                r_cw = hop(comm_cw, send_cw, recv_cw, s, right)
                r_ccw = hop(comm_ccw, send_ccw, recv_ccw, s, left)
                r_cw.start()
                r_ccw.start()

                p_cw = partial(2 - s, c_cw)
                p_ccw = partial(2 + s if s < 2 else 0, c_ccw)

                r_cw.wait()
                r_ccw.wait()

                rs = (s + 1) % 2
                if s < N_DEV - 2:
                    comm_cw[rs] = (
                        comm_cw[rs][...].astype(jnp.float32) + p_cw
                    ).astype(jnp.bfloat16)
                    comm_ccw[rs] = (
                        comm_ccw[rs][...].astype(jnp.float32) + p_ccw
                    ).astype(jnp.bfloat16)
                else:
                    acc_cw[...] = comm_cw[rs][...].astype(jnp.float32) + p_cw
                    acc_ccw[...] = (
                        comm_ccw[rs][...].astype(jnp.float32) + p_ccw
                    )
                    cp0 = pltpu.make_async_copy(
                        acc_cw, out_ref.at[:, pl.ds(c_cw, N_BLK)],
                        out_sems.at[0],
                    )
                    cp1 = pltpu.make_async_copy(
                        acc_ccw, out_ref.at[:, pl.ds(c_ccw, N_BLK)],
                        out_sems.at[1],
                    )
                    cp0.start()
                    cp1.start()
                    if p + 1 < N_PHASES:
                        comm_cw[0] = partial(3, c_cw + 2 * N_BLK).astype(
                            jnp.bfloat16)
                        comm_ccw[0] = partial(1, c_ccw + 2 * N_BLK).astype(
                            jnp.bfloat16)
                    cp0.wait()
                    cp1.wait()

    out = pl.pallas_call(
        body,
        out_shape=jax.ShapeDtypeStruct((M_PER, N_TOTAL), jnp.float32),
        in_specs=[
            pl.BlockSpec(memory_space=pltpu.VMEM),
            pl.BlockSpec(memory_space=pltpu.VMEM),
        ],
        out_specs=pl.BlockSpec(memory_space=pl.ANY),
        scratch_shapes=[
            pltpu.VMEM((2, M_PER, N_BLK), jnp.bfloat16),
            pltpu.VMEM((2, M_PER, N_BLK), jnp.bfloat16),
            pltpu.VMEM((M_PER, N_BLK), jnp.float32),
            pltpu.VMEM((M_PER, N_BLK), jnp.float32),
            pltpu.SemaphoreType.DMA((2,)),
            pltpu.SemaphoreType.DMA((2,)),
            pltpu.SemaphoreType.DMA((2,)),
            pltpu.SemaphoreType.DMA((2,)),
            pltpu.SemaphoreType.DMA((2,)),
        ],
        compiler_params=pltpu.CompilerParams(
            collective_id=0,
            vmem_limit_bytes=42 * 1024 * 1024,
        ),
    )(x16, w16)
    return out


# device time: 358567 ns/iter; 1.9699x vs baseline; 1.0934x over previous
import jax
import jax.numpy as jnp
from jax import lax
from jax.experimental import pallas as pl
from jax.experimental.pallas import tpu as pltpu

N_DEV = 4
M_PER = 1024
N_TOTAL = 8192
N_BLK = 512
N_ROUNDS = 4
CCW0 = N_TOTAL // 2

RECV_OFF_CW = (2, 1, 0)
RECV_OFF_CCW = (2, 3, 0)


def kernel(x, w_mat):
    x16 = x.astype(jnp.bfloat16)
    w16 = w_mat.astype(jnp.bfloat16)

    def body(x_ref, w_ref, out_ref,
             comm_cw, comm_ccw, acc,
             send_cw, recv_cw, send_ccw, recv_ccw, out_sems):
        my_pos = lax.axis_index("i")
        left = lax.rem(my_pos + (N_DEV - 1), N_DEV)
        right = lax.rem(my_pos + 1, N_DEV)

        barrier_sem = pltpu.get_barrier_semaphore()
        for nbr in (left, right):
            pl.semaphore_signal(
                barrier_sem, inc=1,
                device_id=(nbr,), device_id_type=pl.DeviceIdType.MESH,
            )
        pl.semaphore_wait(barrier_sem, 2)

        def partial(off, col0):
            row0 = lax.rem(my_pos + off, N_DEV) * M_PER
            return jnp.dot(
                x_ref[pl.ds(row0, M_PER), :],
                w_ref[:, pl.ds(col0, N_BLK)],
                preferred_element_type=jnp.float32,
            )

        def hop(cw, lane, h):
            comm = comm_cw if cw else comm_ccw
            base = 2 * lane
            return pltpu.make_async_remote_copy(
                src_ref=comm.at[base + h % 2],
                dst_ref=comm.at[base + (h + 1) % 2],
                send_sem=(send_cw if cw else send_ccw).at[base + h % 2],
                recv_sem=(recv_cw if cw else recv_ccw).at[base + (h + 1) % 2],
                device_id=(right if cw else left,),
                device_id_type=pl.DeviceIdType.MESH,
            )

        def cols(cw, r, lane):
            return (0 if cw else CCW0) + (2 * r + lane) * N_BLK

        def add_into(cw, slot, p):
            comm = comm_cw if cw else comm_ccw
            comm[slot] = (comm[slot][...].astype(jnp.float32) + p).astype(
                jnp.bfloat16)

        def load_payload(cw, lane, col0):
            comm = comm_cw if cw else comm_ccw
            off = 3 if cw else 1
            comm[2 * lane] = partial(off, col0).astype(jnp.bfloat16)

        for lane in (0, 1):
            load_payload(True, lane, cols(True, 0, lane))
            load_payload(False, lane, cols(False, 0, lane))

        for r in range(N_ROUNDS):
            c = {(cw, lane): cols(cw, r, lane)
                 for cw in (True, False) for lane in (0, 1)}

            rd = {}
            for lane in (0, 1):
                for cw in (True, False):
                    rd[(cw, lane)] = hop(cw, lane, 0)
                    rd[(cw, lane)].start()

            for h in range(N_DEV - 1):
                last = h == N_DEV - 2
                for lane in (0, 1):
                    p_cw = partial(RECV_OFF_CW[h], c[(True, lane)])
                    p_ccw = partial(RECV_OFF_CCW[h], c[(False, lane)])
                    for cw, pp in ((True, p_cw), (False, p_ccw)):
                        rdma = rd[(cw, lane)]
                        rdma.wait()
                        rslot = 2 * lane + (h + 1) % 2
                        if not last:
                            add_into(cw, rslot, pp)
                            rd[(cw, lane)] = hop(cw, lane, h + 1)
                            rd[(cw, lane)].start()
                        else:
                            comm = comm_cw if cw else comm_ccw
                            a = 2 * lane + (0 if cw else 1)
                            acc[a] = comm[rslot][...].astype(
                                jnp.float32) + pp
                            pltpu.make_async_copy(
                                acc.at[a],
                                out_ref.at[:, pl.ds(c[(cw, lane)], N_BLK)],
                                out_sems.at[a],
                            ).start()

            if r + 1 < N_ROUNDS:
                for lane in (0, 1):
                    load_payload(True, lane, cols(True, r + 1, lane))
                    load_payload(False, lane, cols(False, r + 1, lane))

            for cw in (True, False):
                for lane in (0, 1):
                    a = 2 * lane + (0 if cw else 1)
                    pltpu.make_async_copy(
                        acc.at[a],
                        out_ref.at[:, pl.ds(c[(cw, lane)], N_BLK)],
                        out_sems.at[a],
                    ).wait()

    out = pl.pallas_call(
        body,
        out_shape=jax.ShapeDtypeStruct((M_PER, N_TOTAL), jnp.float32),
        in_specs=[
            pl.BlockSpec(memory_space=pltpu.VMEM),
            pl.BlockSpec(memory_space=pltpu.VMEM),
        ],
        out_specs=pl.BlockSpec(memory_space=pl.ANY),
        scratch_shapes=[
            pltpu.VMEM((4, M_PER, N_BLK), jnp.bfloat16),
            pltpu.VMEM((4, M_PER, N_BLK), jnp.bfloat16),
            pltpu.VMEM((4, M_PER, N_BLK), jnp.float32),
            pltpu.SemaphoreType.DMA((4,)),
            pltpu.SemaphoreType.DMA((4,)),
            pltpu.SemaphoreType.DMA((4,)),
            pltpu.SemaphoreType.DMA((4,)),
            pltpu.SemaphoreType.DMA((4,)),
        ],
        compiler_params=pltpu.CompilerParams(
            collective_id=0,
            vmem_limit_bytes=44 * 1024 * 1024,
        ),
    )(x16, w16)
    return out


# device time: 352569 ns/iter; 2.0034x vs baseline; 1.0170x over previous
import jax
import jax.numpy as jnp
from jax import lax
from jax.experimental import pallas as pl
from jax.experimental.pallas import tpu as pltpu

N_DEV = 4
M_PER = 1024
N_TOTAL = 8192
N_BLK = 512
N_ROUNDS = 4
CCW0 = N_TOTAL // 2

RECV_OFF_CW = (2, 1, 0)
RECV_OFF_CCW = (2, 3, 0)


def kernel(x, w_mat):
    x16 = x.astype(jnp.bfloat16)
    w16 = w_mat.astype(jnp.bfloat16)

    def body(x_ref, w_ref, out_ref,
             comm_cw, comm_ccw, acc,
             send_cw, recv_cw, send_ccw, recv_ccw, out_sems):
        my_pos = lax.axis_index("i")
        left = lax.rem(my_pos + (N_DEV - 1), N_DEV)
        right = lax.rem(my_pos + 1, N_DEV)

        barrier_sem = pltpu.get_barrier_semaphore()
        for nbr in (left, right):
            pl.semaphore_signal(
                barrier_sem, inc=1,
                device_id=(nbr,), device_id_type=pl.DeviceIdType.MESH,
            )
        pl.semaphore_wait(barrier_sem, 2)

        def partial(off, col0):
            row0 = lax.rem(my_pos + off, N_DEV) * M_PER
            return jnp.dot(
                x_ref[pl.ds(row0, M_PER), :],
                w_ref[:, pl.ds(col0, N_BLK)],
                preferred_element_type=jnp.float32,
            )

        def hop(cw, lane, h):
            comm = comm_cw if cw else comm_ccw
            base = 2 * lane
            return pltpu.make_async_remote_copy(
                src_ref=comm.at[base + h % 2],
                dst_ref=comm.at[base + (h + 1) % 2],
                send_sem=(send_cw if cw else send_ccw).at[base + h % 2],
                recv_sem=(recv_cw if cw else recv_ccw).at[base + (h + 1) % 2],
                device_id=(right if cw else left,),
                device_id_type=pl.DeviceIdType.MESH,
            )

        def cols(cw, r, lane):
            return (0 if cw else CCW0) + (2 * r + lane) * N_BLK

        def add_into(cw, slot, p):
            comm = comm_cw if cw else comm_ccw
            comm[slot] = (comm[slot][...].astype(jnp.float32) + p).astype(
                jnp.bfloat16)

        def load_payload(cw, lane, col0):
            comm = comm_cw if cw else comm_ccw
            off = 3 if cw else 1
            comm[2 * lane] = partial(off, col0).astype(jnp.bfloat16)

        def wait_out(r):
            for cw in (True, False):
                for lane in (0, 1):
                    a = 2 * lane + (0 if cw else 1)
                    pltpu.make_async_copy(
                        acc.at[a],
                        out_ref.at[:, pl.ds(cols(cw, r, lane), N_BLK)],
                        out_sems.at[a],
                    ).wait()

        load_payload(True, 0, cols(True, 0, 0))
        load_payload(False, 0, cols(False, 0, 0))

        for r in range(N_ROUNDS):
            c = {(cw, lane): cols(cw, r, lane)
                 for cw in (True, False) for lane in (0, 1)}

            rd = {}
            for cw in (True, False):
                rd[(cw, 0)] = hop(cw, 0, 0)
                rd[(cw, 0)].start()
            if r == 0:
                load_payload(True, 1, c[(True, 1)])
                load_payload(False, 1, c[(False, 1)])
            for cw in (True, False):
                rd[(cw, 1)] = hop(cw, 1, 0)
                rd[(cw, 1)].start()
            if r > 0:
                wait_out(r - 1)

            for h in range(N_DEV - 1):
                last = h == N_DEV - 2
                for lane in (0, 1):
                    p_cw = partial(RECV_OFF_CW[h], c[(True, lane)])
                    p_ccw = partial(RECV_OFF_CCW[h], c[(False, lane)])
                    for cw, pp in ((True, p_cw), (False, p_ccw)):
                        rdma = rd[(cw, lane)]
                        rdma.wait()
                        rslot = 2 * lane + (h + 1) % 2
                        if not last:
                            add_into(cw, rslot, pp)
                            rd[(cw, lane)] = hop(cw, lane, h + 1)
                            rd[(cw, lane)].start()
                        else:
                            comm = comm_cw if cw else comm_ccw
                            a = 2 * lane + (0 if cw else 1)
                            acc[a] = comm[rslot][...].astype(
                                jnp.float32) + pp
                            pltpu.make_async_copy(
                                acc.at[a],
                                out_ref.at[:, pl.ds(c[(cw, lane)], N_BLK)],
                                out_sems.at[a],
                            ).start()

            if r + 1 < N_ROUNDS:
                for lane in (0, 1):
                    load_payload(True, lane, cols(True, r + 1, lane))
                    load_payload(False, lane, cols(False, r + 1, lane))

        wait_out(N_ROUNDS - 1)

    out = pl.pallas_call(
        body,
        out_shape=jax.ShapeDtypeStruct((M_PER, N_TOTAL), jnp.float32),
        in_specs=[
            pl.BlockSpec(memory_space=pltpu.VMEM),
            pl.BlockSpec(memory_space=pltpu.VMEM),
        ],
        out_specs=pl.BlockSpec(memory_space=pl.ANY),
        scratch_shapes=[
            pltpu.VMEM((4, M_PER, N_BLK), jnp.bfloat16),
            pltpu.VMEM((4, M_PER, N_BLK), jnp.bfloat16),
            pltpu.VMEM((4, M_PER, N_BLK), jnp.float32),
            pltpu.SemaphoreType.DMA((4,)),
            pltpu.SemaphoreType.DMA((4,)),
            pltpu.SemaphoreType.DMA((4,)),
            pltpu.SemaphoreType.DMA((4,)),
            pltpu.SemaphoreType.DMA((4,)),
        ],
        compiler_params=pltpu.CompilerParams(
            collective_id=0,
            vmem_limit_bytes=44 * 1024 * 1024,
        ),
    )(x16, w16)
    return out


# device time: 339846 ns/iter; 2.0784x vs baseline; 1.0374x over previous
import jax
import jax.numpy as jnp
from jax import lax
from jax.experimental import pallas as pl
from jax.experimental.pallas import tpu as pltpu

N_DEV = 4
M_PER = 1024
N_TOTAL = 8192
N_BLK = 512
N_ROUNDS = 4
CCW0 = N_TOTAL // 2

RECV_OFF_CW = (2, 1, 0)
RECV_OFF_CCW = (2, 3, 0)


def kernel(x, w_mat):
    x16 = x.astype(jnp.bfloat16)

    def body(x_ref, w_ref, out_ref,
             comm_cw, comm_ccw, acc, w16, wstage,
             send_cw, recv_cw, send_ccw, recv_ccw, out_sems, wsems):
        my_pos = lax.axis_index("i")
        left = lax.rem(my_pos + (N_DEV - 1), N_DEV)
        right = lax.rem(my_pos + 1, N_DEV)

        barrier_sem = pltpu.get_barrier_semaphore()
        for nbr in (left, right):
            pl.semaphore_signal(
                barrier_sem, inc=1,
                device_id=(nbr,), device_id_type=pl.DeviceIdType.MESH,
            )
        pl.semaphore_wait(barrier_sem, 2)

        def cols(cw, r, lane):
            return (0 if cw else CCW0) + (2 * r + lane) * N_BLK

        def wslot(r, cw, lane):
            return (r % 2) * 4 + (0 if cw else 2) + lane

        def fetch_w(r):
            blocks = [(cw, lane) for cw in (True, False) for lane in (0, 1)]
            dmas = {}
            for i, (cw, lane) in enumerate(blocks):
                st = i % 2
                if i >= 2:
                    dmas[i - 2].wait()
                    pcw, plane = blocks[i - 2]
                    w16[wslot(r, pcw, plane)] = (
                        wstage[st][...].astype(jnp.bfloat16))
                d = pltpu.make_async_copy(
                    w_ref.at[:, pl.ds(cols(cw, r, lane), N_BLK)],
                    wstage.at[st], wsems.at[st])
                d.start()
                dmas[i] = d
            for i in (2, 3):
                dmas[i].wait()
                pcw, plane = blocks[i]
                w16[wslot(r, pcw, plane)] = (
                    wstage[i % 2][...].astype(jnp.bfloat16))

        def partial(off, ws):
            row0 = lax.rem(my_pos + off, N_DEV) * M_PER
            return jnp.dot(
                x_ref[pl.ds(row0, M_PER), :],
                w16[ws],
                preferred_element_type=jnp.float32,
            )

        def hop(cw, lane, h):
            comm = comm_cw if cw else comm_ccw
            base = 2 * lane
            return pltpu.make_async_remote_copy(
                src_ref=comm.at[base + h % 2],
                dst_ref=comm.at[base + (h + 1) % 2],
                send_sem=(send_cw if cw else send_ccw).at[base + h % 2],
                recv_sem=(recv_cw if cw else recv_ccw).at[base + (h + 1) % 2],
                device_id=(right if cw else left,),
                device_id_type=pl.DeviceIdType.MESH,
            )

        def add_into(cw, slot, p):
            comm = comm_cw if cw else comm_ccw
            comm[slot] = (comm[slot][...].astype(jnp.float32) + p).astype(
                jnp.bfloat16)

        def load_payload(r, cw, lane):
            comm = comm_cw if cw else comm_ccw
            off = 3 if cw else 1
            comm[2 * lane] = partial(off, wslot(r, cw, lane)).astype(
                jnp.bfloat16)

        def wait_out(r):
            for cw in (True, False):
                for lane in (0, 1):
                    a = 2 * lane + (0 if cw else 1)
                    pltpu.make_async_copy(
                        acc.at[a],
                        out_ref.at[:, pl.ds(cols(cw, r, lane), N_BLK)],
                        out_sems.at[a],
                    ).wait()

        fetch_w(0)
        load_payload(0, True, 0)
        load_payload(0, False, 0)

        for r in range(N_ROUNDS):
            c = {(cw, lane): cols(cw, r, lane)
                 for cw in (True, False) for lane in (0, 1)}

            rd = {}
            for cw in (True, False):
                rd[(cw, 0)] = hop(cw, 0, 0)
                rd[(cw, 0)].start()
            if r == 0:
                load_payload(0, True, 1)
                load_payload(0, False, 1)
            for cw in (True, False):
                rd[(cw, 1)] = hop(cw, 1, 0)
                rd[(cw, 1)].start()
            if r > 0:
                wait_out(r - 1)
            if r + 1 < N_ROUNDS:
                fetch_w(r + 1)

            for h in range(N_DEV - 1):
                last = h == N_DEV - 2
                for lane in (0, 1):
                    p_cw = partial(RECV_OFF_CW[h], wslot(r, True, lane))
                    p_ccw = partial(RECV_OFF_CCW[h], wslot(r, False, lane))
                    for cw, pp in ((True, p_cw), (False, p_ccw)):
                        rdma = rd[(cw, lane)]
                        rdma.wait()
                        rslot = 2 * lane + (h + 1) % 2
                        if not last:
                            add_into(cw, rslot, pp)
                            rd[(cw, lane)] = hop(cw, lane, h + 1)
                            rd[(cw, lane)].start()
                        else:
                            comm = comm_cw if cw else comm_ccw
                            a = 2 * lane + (0 if cw else 1)
                            acc[a] = comm[rslot][...].astype(
                                jnp.float32) + pp
                            pltpu.make_async_copy(
                                acc.at[a],
                                out_ref.at[:, pl.ds(c[(cw, lane)], N_BLK)],
                                out_sems.at[a],
                            ).start()

            if r + 1 < N_ROUNDS:
                for lane in (0, 1):
                    load_payload(r + 1, True, lane)
                    load_payload(r + 1, False, lane)

        wait_out(N_ROUNDS - 1)

    out = pl.pallas_call(
        body,
        out_shape=jax.ShapeDtypeStruct((M_PER, N_TOTAL), jnp.float32),
        in_specs=[
            pl.BlockSpec(memory_space=pltpu.VMEM),
            pl.BlockSpec(memory_space=pl.ANY),
        ],
        out_specs=pl.BlockSpec(memory_space=pl.ANY),
        scratch_shapes=[
            pltpu.VMEM((4, M_PER, N_BLK), jnp.bfloat16),
            pltpu.VMEM((4, M_PER, N_BLK), jnp.bfloat16),
            pltpu.VMEM((4, M_PER, N_BLK), jnp.float32),
            pltpu.VMEM((8, M_PER, N_BLK), jnp.bfloat16),
            pltpu.VMEM((2, M_PER, N_BLK), jnp.float32),
            pltpu.SemaphoreType.DMA((4,)),
            pltpu.SemaphoreType.DMA((4,)),
            pltpu.SemaphoreType.DMA((4,)),
            pltpu.SemaphoreType.DMA((4,)),
            pltpu.SemaphoreType.DMA((4,)),
            pltpu.SemaphoreType.DMA((2,)),
        ],
        compiler_params=pltpu.CompilerParams(
            collective_id=0,
            vmem_limit_bytes=50 * 1024 * 1024,
        ),
    )(x16, w_mat)
    return out


# device time: 319439 ns/iter; 2.2111x vs baseline; 1.0639x over previous
import jax
import jax.numpy as jnp
from jax import lax
from jax.experimental import pallas as pl
from jax.experimental.pallas import tpu as pltpu

N_DEV = 4
M_PER = 1024
N_TOTAL = 8192
N_BLK = 512
N_ROUNDS = 4
CCW0 = N_TOTAL // 2

RECV_OFF_CW = (2, 1, 0)
RECV_OFF_CCW = (2, 3, 0)


def kernel(x, w_mat):
    x16 = x.astype(jnp.bfloat16)

    def body(x_ref, w_ref, out_ref,
             comm_cw, comm_ccw, acc, w16, wstage,
             send_cw, recv_cw, send_ccw, recv_ccw, out_sems, wsems):
        my_pos = lax.axis_index("i")
        left = lax.rem(my_pos + (N_DEV - 1), N_DEV)
        right = lax.rem(my_pos + 1, N_DEV)

        barrier_sem = pltpu.get_barrier_semaphore()
        for nbr in (left, right):
            pl.semaphore_signal(
                barrier_sem, inc=1,
                device_id=(nbr,), device_id_type=pl.DeviceIdType.MESH,
            )
        pl.semaphore_wait(barrier_sem, 2)

        def cols(cw, r, lane):
            return (0 if cw else CCW0) + (2 * r + lane) * N_BLK

        def wslot(r, cw, lane):
            return (r % 2) * 4 + (0 if cw else 2) + lane

        def fetch_w(r):
            blocks = [(cw, lane) for cw in (True, False) for lane in (0, 1)]
            dmas = {}
            for i, (cw, lane) in enumerate(blocks):
                st = i % 2
                if i >= 2:
                    dmas[i - 2].wait()
                    pcw, plane = blocks[i - 2]
                    w16[wslot(r, pcw, plane)] = (
                        wstage[st][...].astype(jnp.bfloat16))
                d = pltpu.make_async_copy(
                    w_ref.at[:, pl.ds(cols(cw, r, lane), N_BLK)],
                    wstage.at[st], wsems.at[st])
                d.start()
                dmas[i] = d
            for i in (2, 3):
                dmas[i].wait()
                pcw, plane = blocks[i]
                w16[wslot(r, pcw, plane)] = (
                    wstage[i % 2][...].astype(jnp.bfloat16))

        def partial(off, ws):
            row0 = lax.rem(my_pos + off, N_DEV) * M_PER
            return jnp.dot(
                x_ref[pl.ds(row0, M_PER), :],
                w16[ws],
                preferred_element_type=jnp.float32,
            )

        def hop(cw, lane, h):
            comm = comm_cw if cw else comm_ccw
            base = 2 * lane
            return pltpu.make_async_remote_copy(
                src_ref=comm.at[base + h % 2],
                dst_ref=comm.at[base + (h + 1) % 2],
                send_sem=(send_cw if cw else send_ccw).at[base + h % 2],
                recv_sem=(recv_cw if cw else recv_ccw).at[base + (h + 1) % 2],
                device_id=(right if cw else left,),
                device_id_type=pl.DeviceIdType.MESH,
            )

        def add_into(cw, slot, p):
            comm = comm_cw if cw else comm_ccw
            comm[slot] = comm[slot][...] + p.astype(jnp.bfloat16)

        def load_payload(r, cw, lane):
            comm = comm_cw if cw else comm_ccw
            off = 3 if cw else 1
            comm[2 * lane] = partial(off, wslot(r, cw, lane)).astype(
                jnp.bfloat16)

        def wait_out(r):
            for cw in (True, False):
                for lane in (0, 1):
                    a = 2 * lane + (0 if cw else 1)
                    pltpu.make_async_copy(
                        acc.at[a],
                        out_ref.at[:, pl.ds(cols(cw, r, lane), N_BLK)],
                        out_sems.at[a],
                    ).wait()

        rd = {}
        fetch_w(0)
        for lane in (0, 1):
            load_payload(0, True, lane)
            load_payload(0, False, lane)
            for cw in (True, False):
                rd[(cw, lane)] = hop(cw, lane, 0)
                rd[(cw, lane)].start()

        for r in range(N_ROUNDS):
            c = {(cw, lane): cols(cw, r, lane)
                 for cw in (True, False) for lane in (0, 1)}

            if r > 0:
                wait_out(r - 1)
            if r + 1 < N_ROUNDS:
                fetch_w(r + 1)

            for h in range(N_DEV - 1):
                last = h == N_DEV - 2
                for lane in (0, 1):
                    p_cw = partial(RECV_OFF_CW[h], wslot(r, True, lane))
                    p_ccw = partial(RECV_OFF_CCW[h], wslot(r, False, lane))
                    for cw, pp in ((True, p_cw), (False, p_ccw)):
                        rdma = rd[(cw, lane)]
                        rdma.wait()
                        rslot = 2 * lane + (h + 1) % 2
                        if not last:
                            add_into(cw, rslot, pp)
                            rd[(cw, lane)] = hop(cw, lane, h + 1)
                            rd[(cw, lane)].start()
                        else:
                            comm = comm_cw if cw else comm_ccw
                            a = 2 * lane + (0 if cw else 1)
                            acc[a] = comm[rslot][...].astype(
                                jnp.float32) + pp
                            pltpu.make_async_copy(
                                acc.at[a],
                                out_ref.at[:, pl.ds(c[(cw, lane)], N_BLK)],
                                out_sems.at[a],
                            ).start()
                    if last and r + 1 < N_ROUNDS:
                        load_payload(r + 1, True, lane)
                        load_payload(r + 1, False, lane)
                        for cw in (True, False):
                            rd[(cw, lane)] = hop(cw, lane, 0)
                            rd[(cw, lane)].start()

        wait_out(N_ROUNDS - 1)

    out = pl.pallas_call(
        body,
        out_shape=jax.ShapeDtypeStruct((M_PER, N_TOTAL), jnp.float32),
        in_specs=[
            pl.BlockSpec(memory_space=pltpu.VMEM),
            pl.BlockSpec(memory_space=pl.ANY),
        ],
        out_specs=pl.BlockSpec(memory_space=pl.ANY),
        scratch_shapes=[
            pltpu.VMEM((4, M_PER, N_BLK), jnp.bfloat16),
            pltpu.VMEM((4, M_PER, N_BLK), jnp.bfloat16),
            pltpu.VMEM((4, M_PER, N_BLK), jnp.float32),
            pltpu.VMEM((8, M_PER, N_BLK), jnp.bfloat16),
            pltpu.VMEM((2, M_PER, N_BLK), jnp.float32),
            pltpu.SemaphoreType.DMA((4,)),
            pltpu.SemaphoreType.DMA((4,)),
            pltpu.SemaphoreType.DMA((4,)),
            pltpu.SemaphoreType.DMA((4,)),
            pltpu.SemaphoreType.DMA((4,)),
            pltpu.SemaphoreType.DMA((2,)),
        ],
        compiler_params=pltpu.CompilerParams(
            collective_id=0,
            vmem_limit_bytes=50 * 1024 * 1024,
        ),
    )(x16, w_mat)
    return out
